# baseline (speedup 1.0000x reference)
import functools

import jax
import jax.numpy as jnp
from jax import lax
from jax.experimental import pallas as pl
from jax.experimental.pallas import tpu as pltpu
from jax.experimental.pallas import tpu_sc as plsc

_NUM_CORES = 2
_NUM_SUBCORES = 16
_LANES = 16
_NW = _NUM_CORES * _NUM_SUBCORES

_B = 1024
_F = 128
_E = 256
_CHUNKS = _E // _LANES
_ROWS_PER_W = _B // _NW

_K = 1
_EMA_D = 0.95 ** (1.0 / 30000)


def _sc_body(r_hbm, out_hbm, buf, hist, sem0, sem1):
    wid = lax.axis_index("c") * _NUM_SUBCORES + lax.axis_index("s")
    base = wid * _ROWS_PER_W
    sems = (sem0, sem1)

    zero = jnp.zeros((_LANES,), jnp.float32)
    for g in range(_F // _LANES):
        hist[pl.ds(g * _LANES, _LANES)] = zero

    lane = lax.iota(jnp.int32, _LANES)
    lane0 = lane == 0
    ones_v = jnp.ones((_LANES,), jnp.float32)

    cp = pltpu.make_async_copy(r_hbm.at[base], buf.at[0], sems[0])
    cp.start()
    pending = cp
    for i in range(_ROWS_PER_W):
        slot = i % 2
        if i + 1 < _ROWS_PER_W:
            nslot = (i + 1) % 2
            ncp = pltpu.make_async_copy(
                r_hbm.at[base + i + 1], buf.at[nslot], sems[nslot])
            ncp.start()
        pending.wait()

        def feat_body(f, carry, _slot=slot):
            bv, bi = carry
            acc = buf[_slot, f * _CHUNKS]
            for e in range(1, _CHUNKS):
                acc = acc + buf[_slot, f * _CHUNKS + e]
            tot = jnp.sum(acc)
            better = tot > bv
            bv = jnp.where(better, tot, bv)
            bi = jnp.where(better, f, bi)
            return bv, bi

        _, best_idx = lax.fori_loop(
            0, _F, feat_body,
            (jnp.float32(-jnp.inf), jnp.int32(0)))

        idxv = jnp.full((_LANES,), best_idx, dtype=jnp.int32)
        plsc.addupdate_scatter(hist, [idxv], ones_v, mask=lane0)

        if i + 1 < _ROWS_PER_W:
            pending = ncp

    pltpu.sync_copy(hist, out_hbm.at[wid])


_sc_win_hist = functools.partial(
    pl.kernel,
    out_type=jax.ShapeDtypeStruct((_NW, _F), jnp.float32),
    mesh=plsc.VectorSubcoreMesh(
        core_axis_name="c", subcore_axis_name="s",
        num_cores=_NUM_CORES, num_subcores=_NUM_SUBCORES),
    scratch_types=[
        pltpu.VMEM((2, _F * _CHUNKS, _LANES), jnp.float32),
        pltpu.VMEM((_F,), jnp.float32),
        pltpu.SemaphoreType.DMA,
        pltpu.SemaphoreType.DMA,
    ],
    compiler_params=pltpu.CompilerParams(
        needs_layout_passes=False, use_tc_tiling_on_sc=False),
)(_sc_body)


def _finish_body(p_ref, freq_ref, boost_ref):
    counts = jnp.sum(p_ref[...], axis=0)
    freq = counts * (1.0 / float(_K * _B))
    freq_ref[...] = _EMA_D * freq + (1.0 - _EMA_D) * freq
    boost_ref[...] = jnp.ones((_F,), jnp.float32)


_finish = pl.pallas_call(
    _finish_body,
    out_shape=(
        jax.ShapeDtypeStruct((_F,), jnp.float32),
        jax.ShapeDtypeStruct((_F,), jnp.float32),
    ),
)


def kernel(R):
    r = R.reshape(_B, _F * _CHUNKS, _LANES)
    partials = _sc_win_hist(r)
    freq_ema, boosting_weights = _finish(partials)
    return freq_ema, boosting_weights
